# Initial kernel scaffold; baseline (speedup 1.0000x reference)
import jax
import jax.numpy as jnp
from jax.experimental import pallas as pl


def kernel(inputs, split_values, split_index_array, leaf_classes_array, training):
    raise NotImplementedError("write your pallas kernel here")


# speedup vs baseline: 1.8780x; 1.8780x over previous
import jax
import jax.numpy as jnp
import numpy as _np
from jax import lax
from jax.experimental import pallas as pl
from jax.experimental.pallas import tpu as pltpu
from jax.experimental.pallas import tpu_sc as plsc

DEPTH = 6
E = 128
F = 64
B = 512
LEAVES = 64
NODES = 63
NC, NS, LANES = 2, 16, 16
NW = NC * NS
EPW = E // NW
ROWS = EPW * NODES
G_FULL = ROWS // LANES
G_REM = ROWS - G_FULL * LANES
PAD = LANES
_Z0 = float(_np.uint32(0x34B17218).view(_np.float32))


def _argmax_group(sia_buf, sv_buf, ridx):
    m0 = jnp.full((LANES,), -jnp.inf, jnp.float32)
    a0 = jnp.zeros((LANES,), jnp.int32)

    def feat(fi, carry):
        m, a = carry
        col = jnp.full((LANES,), fi, jnp.int32)
        v = plsc.load_gather(sia_buf, [ridx, col])
        upd = v > m
        return jnp.where(upd, v, m), jnp.where(upd, fi, a)

    m, a = lax.fori_loop(0, F, feat, (m0, a0))
    t = plsc.load_gather(sv_buf, [ridx, a])
    return a, t


def _sc_body(sia_hbm, sv_hbm, x_hbm, lc_hbm, out_hbm,
             sia_buf, sv_buf, f_loc, t_loc, x_buf, lc_buf, out_buf):
    cid = lax.axis_index("c")
    sid = lax.axis_index("s")
    wid = sid * NC + cid
    row0 = wid * ROWS
    iota = lax.broadcasted_iota(jnp.int32, (LANES,), 0)

    def group_a(g, _):
        pltpu.sync_copy(sia_hbm.at[pl.ds(row0 + g * LANES, LANES), :], sia_buf)
        pltpu.sync_copy(sv_hbm.at[pl.ds(row0 + g * LANES, LANES), :], sv_buf)
        a, t = _argmax_group(sia_buf, sv_buf, iota)
        f_loc[pl.ds(PAD + g * LANES, LANES)] = a
        t_loc[pl.ds(PAD + g * LANES, LANES)] = t
        return 0

    lax.fori_loop(0, G_FULL, group_a, 0)

    base_rem = row0 + ROWS - LANES
    pltpu.sync_copy(sia_hbm.at[pl.ds(base_rem, LANES), :], sia_buf)
    pltpu.sync_copy(sv_hbm.at[pl.ds(base_rem, LANES), :], sv_buf)
    ridx = jnp.minimum(iota, G_REM - 1) + (LANES - G_REM)
    a, t = _argmax_group(sia_buf, sv_buf, ridx)
    f_loc[pl.ds(PAD + G_FULL * LANES, LANES)] = a
    t_loc[pl.ds(PAD + G_FULL * LANES, LANES)] = t

    pltpu.sync_copy(lc_hbm.at[pl.ds(wid * EPW, EPW), :], lc_buf)

    def chunk_b(c, _):
        pltpu.sync_copy(x_hbm.at[pl.ds(c * LANES, LANES), :], x_buf)
        for el in range(EPW):
            off = jnp.full((LANES,), PAD + el * NODES, jnp.int32)
            n = jnp.zeros((LANES,), jnp.int32)
            for _d in range(DEPTH):
                idx = n + off
                fsel = plsc.load_gather(f_loc, [idx])
                tsel = plsc.load_gather(t_loc, [idx])
                x = plsc.load_gather(x_buf, [iota, fsel])
                bit = jnp.where(tsel - x <= _Z0, 1, 0).astype(jnp.int32)
                n = 2 * n + 1 + bit
            leaf = n - NODES
            erow = jnp.full((LANES,), el, jnp.int32)
            val = plsc.load_gather(lc_buf, [erow, leaf])
            out_buf[el, pl.ds(c * LANES, LANES)] = val
        return 0

    lax.fori_loop(0, B // LANES, chunk_b, 0)
    pltpu.sync_copy(out_buf, out_hbm.at[pl.ds(wid * EPW, EPW), :])


def kernel(inputs, split_values, split_index_array, leaf_classes_array, training):
    del training
    inputs = inputs.astype(jnp.bfloat16).astype(jnp.float32)
    sia = split_index_array.reshape(E * NODES, F)
    sv = split_values.reshape(E * NODES, F)
    mesh = plsc.VectorSubcoreMesh(core_axis_name="c", subcore_axis_name="s",
                                  num_cores=NC, num_subcores=NS)
    run = pl.kernel(
        _sc_body,
        out_type=jax.ShapeDtypeStruct((E, B), jnp.float32),
        mesh=mesh,
        compiler_params=pltpu.CompilerParams(use_tc_tiling_on_sc=False,
                                             needs_layout_passes=False),
        scratch_types=[
            pltpu.VMEM((LANES, F), jnp.float32),
            pltpu.VMEM((LANES, F), jnp.float32),
            pltpu.VMEM((PAD + G_FULL * LANES + LANES,), jnp.int32),
            pltpu.VMEM((PAD + G_FULL * LANES + LANES,), jnp.float32),
            pltpu.VMEM((LANES, F), jnp.float32),
            pltpu.VMEM((EPW, LEAVES), jnp.float32),
            pltpu.VMEM((EPW, B), jnp.float32),
        ],
    )
    out_eb = run(sia, sv, inputs, leaf_classes_array)
    return out_eb.T


# speedup vs baseline: 2.9211x; 1.5555x over previous
import jax
import jax.numpy as jnp
import numpy as _np
from jax import lax
from jax.experimental import pallas as pl
from jax.experimental.pallas import tpu as pltpu
from jax.experimental.pallas import tpu_sc as plsc

DEPTH = 6
E = 128
F = 64
B = 512
LEAVES = 64
NODES = 63
NC, NS, LANES = 2, 16, 16
NW = NC * NS
EPW = E // NW
ROWS = EPW * NODES
NG = ROWS // LANES + 1
G_REM = ROWS - (NG - 1) * LANES
PAD = LANES
_Z0 = float(_np.uint32(0x34B17218).view(_np.float32))
NCHAIN = 4


def _argmax_group(sia_buf, sv_buf, ridx):
    span = F // NCHAIN
    ms, as_ = [], []
    for c in range(NCHAIN):
        m = jnp.full((LANES,), -jnp.inf, jnp.float32)
        a = jnp.zeros((LANES,), jnp.int32)
        for j in range(span):
            fi = c * span + j
            col = jnp.full((LANES,), fi, jnp.int32)
            v = plsc.load_gather(sia_buf, [ridx, col])
            upd = v > m
            m = jnp.where(upd, v, m)
            a = jnp.where(upd, fi, a)
        ms.append(m)
        as_.append(a)
    while len(ms) > 1:
        nm, na = [], []
        for i in range(0, len(ms), 2):
            upd = ms[i + 1] > ms[i]
            nm.append(jnp.where(upd, ms[i + 1], ms[i]))
            na.append(jnp.where(upd, as_[i + 1], as_[i]))
        ms, as_ = nm, na
    t = plsc.load_gather(sv_buf, [ridx, as_[0]])
    return as_[0], t


def _sc_body(sia_hbm, sv_hbm, x_hbm, lc_hbm, out_hbm,
             sia2, sv2, f_loc, t_loc, x_all, lc_buf, out_buf,
             sem_a0, sem_a1, sem_x, sem_l):
    cid = lax.axis_index("c")
    sid = lax.axis_index("s")
    wid = sid * NC + cid
    row0 = wid * ROWS
    iota = lax.broadcasted_iota(jnp.int32, (LANES,), 0)
    sems = [sem_a0, sem_a1]

    def g_base(g):
        return row0 + jnp.minimum(g * LANES, ROWS - LANES)

    def start_group(g, p):
        pltpu.async_copy(sia_hbm.at[pl.ds(g_base(g), LANES), :], sia2.at[p], sems[p])
        pltpu.async_copy(sv_hbm.at[pl.ds(g_base(g), LANES), :], sv2.at[p], sems[p])

    def wait_group(p):
        pltpu.make_async_copy(sia_hbm.at[pl.ds(0, LANES), :], sia2.at[p], sems[p]).wait()
        pltpu.make_async_copy(sv_hbm.at[pl.ds(0, LANES), :], sv2.at[p], sems[p]).wait()

    start_group(0, 0)
    start_group(1, 1)
    cx = pltpu.async_copy(x_hbm, x_all, sem_x)
    cl = pltpu.async_copy(lc_hbm.at[pl.ds(wid * EPW, EPW), :], lc_buf, sem_l)

    @pl.loop(0, NG - 2, step=2)
    def _(g0):
        for p in range(2):
            g = g0 + p
            wait_group(p)
            a, t = _argmax_group(sia2.at[p], sv2.at[p], iota)
            f_loc[pl.ds(PAD + g * LANES, LANES)] = a
            t_loc[pl.ds(PAD + g * LANES, LANES)] = t
            start_group(g + 2, p)

    wait_group(0)
    a, t = _argmax_group(sia2.at[0], sv2.at[0], iota)
    f_loc[pl.ds(PAD + (NG - 2) * LANES, LANES)] = a
    t_loc[pl.ds(PAD + (NG - 2) * LANES, LANES)] = t
    wait_group(1)
    ridx = jnp.minimum(iota, G_REM - 1) + (LANES - G_REM)
    a, t = _argmax_group(sia2.at[1], sv2.at[1], ridx)
    f_loc[pl.ds(PAD + (NG - 1) * LANES, LANES)] = a
    t_loc[pl.ds(PAD + (NG - 1) * LANES, LANES)] = t

    cx.wait()
    cl.wait()

    @pl.loop(0, B // LANES)
    def _(c):
        row = c * LANES + iota
        for el in range(EPW):
            off = jnp.full((LANES,), PAD + el * NODES, jnp.int32)
            n = jnp.zeros((LANES,), jnp.int32)
            for _d in range(DEPTH):
                idx = n + off
                fsel = plsc.load_gather(f_loc, [idx])
                tsel = plsc.load_gather(t_loc, [idx])
                x = plsc.load_gather(x_all, [row, fsel])
                bit = jnp.where(tsel - x <= _Z0, 1, 0).astype(jnp.int32)
                n = 2 * n + 1 + bit
            leaf = n - NODES
            erow = jnp.full((LANES,), el, jnp.int32)
            val = plsc.load_gather(lc_buf, [erow, leaf])
            out_buf[el, pl.ds(c * LANES, LANES)] = val

    pltpu.sync_copy(out_buf, out_hbm.at[pl.ds(wid * EPW, EPW), :])


def kernel(inputs, split_values, split_index_array, leaf_classes_array, training):
    del training
    inputs = inputs.astype(jnp.bfloat16).astype(jnp.float32)
    sia = split_index_array.reshape(E * NODES, F)
    sv = split_values.reshape(E * NODES, F)
    mesh = plsc.VectorSubcoreMesh(core_axis_name="c", subcore_axis_name="s",
                                  num_cores=NC, num_subcores=NS)
    run = pl.kernel(
        _sc_body,
        out_type=jax.ShapeDtypeStruct((E, B), jnp.float32),
        mesh=mesh,
        compiler_params=pltpu.CompilerParams(use_tc_tiling_on_sc=False,
                                             needs_layout_passes=False),
        scratch_types=[
            pltpu.VMEM((2, LANES, F), jnp.float32),
            pltpu.VMEM((2, LANES, F), jnp.float32),
            pltpu.VMEM((PAD + NG * LANES,), jnp.int32),
            pltpu.VMEM((PAD + NG * LANES,), jnp.float32),
            pltpu.VMEM((B, F), jnp.float32),
            pltpu.VMEM((EPW, LEAVES), jnp.float32),
            pltpu.VMEM((EPW, B), jnp.float32),
            pltpu.SemaphoreType.DMA,
            pltpu.SemaphoreType.DMA,
            pltpu.SemaphoreType.DMA,
            pltpu.SemaphoreType.DMA,
        ],
    )
    out_eb = run(sia, sv, inputs, leaf_classes_array)
    return out_eb.T


# speedup vs baseline: 3.4035x; 1.1651x over previous
import jax
import jax.numpy as jnp
import numpy as _np
from jax import lax
from jax.experimental import pallas as pl
from jax.experimental.pallas import tpu as pltpu
from jax.experimental.pallas import tpu_sc as plsc

DEPTH = 6
E = 128
F = 64
B = 512
LEAVES = 64
NODES = 63
NC, NS, LANES = 2, 16, 16
NW = NC * NS
EPW = E // NW
ROWS = EPW * NODES
NG = ROWS // LANES + 1
G_REM = ROWS - (NG - 1) * LANES
PAD = LANES
_Z0 = float(_np.uint32(0x34B17218).view(_np.float32))
NCHAIN = 4


def _argmax_group(sia_buf, sv_buf, ridx):
    span = F // NCHAIN
    ms, as_ = [], []
    for c in range(NCHAIN):
        m = jnp.full((LANES,), -jnp.inf, jnp.float32)
        a = jnp.zeros((LANES,), jnp.int32)
        for j in range(span):
            fi = c * span + j
            col = jnp.full((LANES,), fi, jnp.int32)
            v = plsc.load_gather(sia_buf, [ridx, col])
            upd = v > m
            m = jnp.where(upd, v, m)
            a = jnp.where(upd, fi, a)
        ms.append(m)
        as_.append(a)
    while len(ms) > 1:
        nm, na = [], []
        for i in range(0, len(ms), 2):
            upd = ms[i + 1] > ms[i]
            nm.append(jnp.where(upd, ms[i + 1], ms[i]))
            na.append(jnp.where(upd, as_[i + 1], as_[i]))
        ms, as_ = nm, na
    t = plsc.load_gather(sv_buf, [ridx, as_[0]])
    return as_[0], t


def _sc_body(sia_hbm, sv_hbm, x_hbm, lc_hbm, out_hbm,
             sia2, sv2, f_loc, t_loc, x_all, lc_buf, out_buf,
             sem_a0, sem_a1, sem_x, sem_l):
    cid = lax.axis_index("c")
    sid = lax.axis_index("s")
    wid = sid * NC + cid
    row0 = wid * ROWS
    iota = lax.broadcasted_iota(jnp.int32, (LANES,), 0)
    sems = [sem_a0, sem_a1]

    def g_base(g):
        return row0 + jnp.minimum(g * LANES, ROWS - LANES)

    def start_group(g, p):
        pltpu.async_copy(sia_hbm.at[pl.ds(g_base(g), LANES), :], sia2.at[p], sems[p])
        pltpu.async_copy(sv_hbm.at[pl.ds(g_base(g), LANES), :], sv2.at[p], sems[p])

    def wait_group(p):
        pltpu.make_async_copy(sia_hbm.at[pl.ds(0, LANES), :], sia2.at[p], sems[p]).wait()
        pltpu.make_async_copy(sv_hbm.at[pl.ds(0, LANES), :], sv2.at[p], sems[p]).wait()

    start_group(0, 0)
    start_group(1, 1)
    cx = pltpu.async_copy(x_hbm, x_all, sem_x)
    cl = pltpu.async_copy(lc_hbm.at[pl.ds(wid * EPW, EPW), :], lc_buf, sem_l)

    @pl.loop(0, NG - 2, step=2)
    def _(g0):
        for p in range(2):
            g = g0 + p
            wait_group(p)
            a, t = _argmax_group(sia2.at[p], sv2.at[p], iota)
            f_loc[pl.ds(PAD + g * LANES, LANES)] = a
            t_loc[pl.ds(PAD + g * LANES, LANES)] = t
            start_group(g + 2, p)

    wait_group(0)
    a, t = _argmax_group(sia2.at[0], sv2.at[0], iota)
    f_loc[pl.ds(PAD + (NG - 2) * LANES, LANES)] = a
    t_loc[pl.ds(PAD + (NG - 2) * LANES, LANES)] = t
    wait_group(1)
    ridx = jnp.minimum(iota, G_REM - 1) + (LANES - G_REM)
    a, t = _argmax_group(sia2.at[1], sv2.at[1], ridx)
    f_loc[pl.ds(PAD + (NG - 1) * LANES, LANES)] = a
    t_loc[pl.ds(PAD + (NG - 1) * LANES, LANES)] = t

    cx.wait()
    cl.wait()

    @plsc.parallel_loop(0, B // LANES, step=1, unroll=2)
    def _(c):
        row = c * LANES + iota
        for el in range(EPW):
            off = jnp.full((LANES,), PAD + el * NODES, jnp.int32)
            n = jnp.zeros((LANES,), jnp.int32)
            for _d in range(DEPTH):
                idx = n + off
                fsel = plsc.load_gather(f_loc, [idx])
                tsel = plsc.load_gather(t_loc, [idx])
                x = plsc.load_gather(x_all, [row, fsel])
                bit = jnp.where(tsel - x <= _Z0, 1, 0).astype(jnp.int32)
                n = 2 * n + 1 + bit
            leaf = n - NODES
            erow = jnp.full((LANES,), el, jnp.int32)
            val = plsc.load_gather(lc_buf, [erow, leaf])
            out_buf[el, pl.ds(c * LANES, LANES)] = val

    pltpu.sync_copy(out_buf, out_hbm.at[pl.ds(wid * EPW, EPW), :])


def kernel(inputs, split_values, split_index_array, leaf_classes_array, training):
    del training
    inputs = inputs.astype(jnp.bfloat16).astype(jnp.float32)
    sia = split_index_array.reshape(E * NODES, F)
    sv = split_values.reshape(E * NODES, F)
    mesh = plsc.VectorSubcoreMesh(core_axis_name="c", subcore_axis_name="s",
                                  num_cores=NC, num_subcores=NS)
    run = pl.kernel(
        _sc_body,
        out_type=jax.ShapeDtypeStruct((E, B), jnp.float32),
        mesh=mesh,
        compiler_params=pltpu.CompilerParams(use_tc_tiling_on_sc=False,
                                             needs_layout_passes=False),
        scratch_types=[
            pltpu.VMEM((2, LANES, F), jnp.float32),
            pltpu.VMEM((2, LANES, F), jnp.float32),
            pltpu.VMEM((PAD + NG * LANES,), jnp.int32),
            pltpu.VMEM((PAD + NG * LANES,), jnp.float32),
            pltpu.VMEM((B, F), jnp.float32),
            pltpu.VMEM((EPW, LEAVES), jnp.float32),
            pltpu.VMEM((EPW, B), jnp.float32),
            pltpu.SemaphoreType.DMA,
            pltpu.SemaphoreType.DMA,
            pltpu.SemaphoreType.DMA,
            pltpu.SemaphoreType.DMA,
        ],
    )
    out_eb = run(sia, sv, inputs, leaf_classes_array)
    return out_eb.T


# speedup vs baseline: 3.4179x; 1.0042x over previous
import jax
import jax.numpy as jnp
import numpy as _np
from jax import lax
from jax.experimental import pallas as pl
from jax.experimental.pallas import tpu as pltpu
from jax.experimental.pallas import tpu_sc as plsc

DEPTH = 6
E = 128
F = 64
B = 512
LEAVES = 64
NODES = 63
NC, NS, LANES = 2, 16, 16
NW = NC * NS
EPW = E // NW
ROWS = EPW * NODES
NG = ROWS // LANES + 1
G_REM = ROWS - (NG - 1) * LANES
PAD = LANES
_Z0 = float(_np.uint32(0x34B17218).view(_np.float32))
NCHAIN = 4


def _argmax_group(sia_buf, sv_buf, ridx):
    span = F // NCHAIN
    ms, as_ = [], []
    for c in range(NCHAIN):
        m = jnp.full((LANES,), -jnp.inf, jnp.float32)
        a = jnp.zeros((LANES,), jnp.int32)
        for j in range(span):
            fi = c * span + j
            col = jnp.full((LANES,), fi, jnp.int32)
            v = plsc.load_gather(sia_buf, [ridx, col])
            upd = v > m
            m = jnp.where(upd, v, m)
            a = jnp.where(upd, fi, a)
        ms.append(m)
        as_.append(a)
    while len(ms) > 1:
        nm, na = [], []
        for i in range(0, len(ms), 2):
            upd = ms[i + 1] > ms[i]
            nm.append(jnp.where(upd, ms[i + 1], ms[i]))
            na.append(jnp.where(upd, as_[i + 1], as_[i]))
        ms, as_ = nm, na
    t = plsc.load_gather(sv_buf, [ridx, as_[0]])
    return as_[0], t


def _sc_body(sia_hbm, sv_hbm, x_hbm, lc_hbm, out_hbm,
             sia2, sv2, f_loc, t_loc, x_all, lc_buf, out_buf,
             sem_a0, sem_a1, sem_x, sem_l):
    cid = lax.axis_index("c")
    sid = lax.axis_index("s")
    wid = sid * NC + cid
    row0 = wid * ROWS
    iota = lax.broadcasted_iota(jnp.int32, (LANES,), 0)
    sems = [sem_a0, sem_a1]

    def g_base(g):
        return row0 + jnp.minimum(g * LANES, ROWS - LANES)

    def start_group(g, p):
        pltpu.async_copy(sia_hbm.at[pl.ds(g_base(g), LANES), :], sia2.at[p], sems[p])
        pltpu.async_copy(sv_hbm.at[pl.ds(g_base(g), LANES), :], sv2.at[p], sems[p])

    def wait_group(p):
        pltpu.make_async_copy(sia_hbm.at[pl.ds(0, LANES), :], sia2.at[p], sems[p]).wait()
        pltpu.make_async_copy(sv_hbm.at[pl.ds(0, LANES), :], sv2.at[p], sems[p]).wait()

    start_group(0, 0)
    start_group(1, 1)
    cx = pltpu.async_copy(x_hbm, x_all, sem_x)
    cl = pltpu.async_copy(lc_hbm.at[pl.ds(wid * EPW, EPW), :], lc_buf, sem_l)

    @pl.loop(0, NG - 2, step=2)
    def _(g0):
        for p in range(2):
            g = g0 + p
            wait_group(p)
            a, t = _argmax_group(sia2.at[p], sv2.at[p], iota)
            f_loc[pl.ds(PAD + g * LANES, LANES)] = a
            t_loc[pl.ds(PAD + g * LANES, LANES)] = t
            start_group(g + 2, p)

    wait_group(0)
    a, t = _argmax_group(sia2.at[0], sv2.at[0], iota)
    f_loc[pl.ds(PAD + (NG - 2) * LANES, LANES)] = a
    t_loc[pl.ds(PAD + (NG - 2) * LANES, LANES)] = t
    wait_group(1)
    ridx = jnp.minimum(iota, G_REM - 1) + (LANES - G_REM)
    a, t = _argmax_group(sia2.at[1], sv2.at[1], ridx)
    f_loc[pl.ds(PAD + (NG - 1) * LANES, LANES)] = a
    t_loc[pl.ds(PAD + (NG - 1) * LANES, LANES)] = t

    cx.wait()
    cl.wait()

    @plsc.parallel_loop(0, B // LANES, step=1, unroll=4)
    def _(c):
        row = c * LANES + iota
        for el in range(EPW):
            off = jnp.full((LANES,), PAD + el * NODES, jnp.int32)
            n = jnp.zeros((LANES,), jnp.int32)
            for _d in range(DEPTH):
                idx = n + off
                fsel = plsc.load_gather(f_loc, [idx])
                tsel = plsc.load_gather(t_loc, [idx])
                x = plsc.load_gather(x_all, [row, fsel])
                bit = jnp.where(tsel - x <= _Z0, 1, 0).astype(jnp.int32)
                n = 2 * n + 1 + bit
            leaf = n - NODES
            erow = jnp.full((LANES,), el, jnp.int32)
            val = plsc.load_gather(lc_buf, [erow, leaf])
            out_buf[el, pl.ds(c * LANES, LANES)] = val

    pltpu.sync_copy(out_buf, out_hbm.at[pl.ds(wid * EPW, EPW), :])


def kernel(inputs, split_values, split_index_array, leaf_classes_array, training):
    del training
    inputs = inputs.astype(jnp.bfloat16).astype(jnp.float32)
    sia = split_index_array.reshape(E * NODES, F)
    sv = split_values.reshape(E * NODES, F)
    mesh = plsc.VectorSubcoreMesh(core_axis_name="c", subcore_axis_name="s",
                                  num_cores=NC, num_subcores=NS)
    run = pl.kernel(
        _sc_body,
        out_type=jax.ShapeDtypeStruct((E, B), jnp.float32),
        mesh=mesh,
        compiler_params=pltpu.CompilerParams(use_tc_tiling_on_sc=False,
                                             needs_layout_passes=False),
        scratch_types=[
            pltpu.VMEM((2, LANES, F), jnp.float32),
            pltpu.VMEM((2, LANES, F), jnp.float32),
            pltpu.VMEM((PAD + NG * LANES,), jnp.int32),
            pltpu.VMEM((PAD + NG * LANES,), jnp.float32),
            pltpu.VMEM((B, F), jnp.float32),
            pltpu.VMEM((EPW, LEAVES), jnp.float32),
            pltpu.VMEM((EPW, B), jnp.float32),
            pltpu.SemaphoreType.DMA,
            pltpu.SemaphoreType.DMA,
            pltpu.SemaphoreType.DMA,
            pltpu.SemaphoreType.DMA,
        ],
    )
    out_eb = run(sia, sv, inputs, leaf_classes_array)
    return out_eb.T
